# Initial kernel scaffold; baseline (speedup 1.0000x reference)
import jax
import jax.numpy as jnp
from jax.experimental import pallas as pl


def kernel(x, W):
    raise NotImplementedError("write your pallas kernel here")


# speedup vs baseline: 3.2462x; 3.2462x over previous
import jax
import jax.numpy as jnp
from jax import lax
from jax.experimental import pallas as pl
from jax.experimental.pallas import tpu as pltpu

E = 8
K = 256
D = 2048
H = 2048
B = 4
HT = 256


def _topk_loss_kernel(v_ref, w_ref, i_ref, loss_ref, c_ref):
    v = v_ref[0]
    iota_d = lax.broadcasted_iota(jnp.int32, (E, D), 1)
    iota_k = lax.broadcasted_iota(jnp.int32, (E, K), 1)

    def step(kk, carry):
        vals, idxs, work = carry
        m = jnp.max(work, axis=1, keepdims=True)
        am = jnp.min(jnp.where(work == m, iota_d, D), axis=1, keepdims=True)
        vals = jnp.where(iota_k == kk, m, vals)
        idxs = jnp.where(iota_k == kk, am, idxs)
        work = jnp.where(iota_d == am, -jnp.inf, work)
        return vals, idxs, work

    vals0 = jnp.zeros((E, K), jnp.float32)
    idxs0 = jnp.zeros((E, K), jnp.int32)
    vals, idxs, work = lax.fori_loop(0, K, step, (vals0, idxs0, v))
    w_ref[0] = vals
    i_ref[0] = idxs

    chosen = jnp.where(work == -jnp.inf, 1.0, 0.0).astype(jnp.float32)
    b = pl.program_id(0)

    @pl.when(b == 0)
    def _():
        c_ref[...] = chosen

    @pl.when(b > 0)
    def _():
        c_ref[...] = c_ref[...] + chosen

    @pl.when(b == B - 1)
    def _():
        u = c_ref[...] * (1.0 / (B * K + 1e-9)) - (1.0 / E)
        loss_ref[...] = (jnp.sum(u * u) * (1.0 / (E * D))).reshape(1, 1)


def _gather_kernel(x_ref, i_ref, out_ref):
    xb = x_ref[0]
    row = i_ref[0, 0, :].reshape(1, K)
    for dc in range(D // K):
        iota = lax.broadcasted_iota(jnp.int32, (K, K), 0) + dc * K
        p = (iota == row).astype(jnp.float32)
        part = lax.dot_general(
            xb[:, dc * K:(dc + 1) * K], p,
            (((1,), (0,)), ((), ())),
            preferred_element_type=jnp.float32,
        )
        if dc == 0:
            out_ref[0, 0] = part
        else:
            out_ref[0, 0] = out_ref[0, 0] + part


def kernel(x, W):
    xt = jnp.swapaxes(x, -1, -2)
    router_logit = jnp.einsum('bdh,eh->bde', xt, W)
    router_logit = jax.nn.softmax(router_logit, axis=-1)
    noise = jax.random.normal(jax.random.key(1234), router_logit.shape,
                              dtype=router_logit.dtype) * 0.001
    v = router_logit + noise
    vt = jnp.swapaxes(v, 1, 2)

    w_bek, i_bek, loss = pl.pallas_call(
        _topk_loss_kernel,
        grid=(B,),
        in_specs=[pl.BlockSpec((1, E, D), lambda b: (b, 0, 0))],
        out_specs=[
            pl.BlockSpec((1, E, K), lambda b: (b, 0, 0)),
            pl.BlockSpec((1, E, K), lambda b: (b, 0, 0)),
            pl.BlockSpec((1, 1), lambda b: (0, 0)),
        ],
        out_shape=[
            jax.ShapeDtypeStruct((B, E, K), jnp.float32),
            jax.ShapeDtypeStruct((B, E, K), jnp.int32),
            jax.ShapeDtypeStruct((1, 1), jnp.float32),
        ],
        scratch_shapes=[pltpu.VMEM((E, D), jnp.float32)],
    )(vt)

    idx_flat = i_bek.reshape(B * E, 1, K)
    tokens = pl.pallas_call(
        _gather_kernel,
        grid=(B, H // HT, E),
        in_specs=[
            pl.BlockSpec((1, HT, D), lambda b, h, e: (b, h, 0)),
            pl.BlockSpec((1, 1, K), lambda b, h, e: (b * E + e, 0, 0)),
        ],
        out_specs=pl.BlockSpec((1, 1, HT, K), lambda b, h, e: (e, b, h, 0)),
        out_shape=jax.ShapeDtypeStruct((E, B, H, K), jnp.float32),
    )(x, idx_flat)

    weights = jnp.transpose(w_bek, (1, 0, 2))
    indices = jnp.transpose(i_bek, (1, 0, 2))
    return tokens, weights, indices, loss.reshape(())
